# Initial kernel scaffold; baseline (speedup 1.0000x reference)
import functools

import jax
import jax.numpy as jnp
from jax import lax
from jax.experimental import pallas as pl
from jax.experimental.pallas import tpu as pltpu
from jax.experimental.pallas import tpu_sc as plsc

N = 10000
E = 320000
D = 128
NC, NS = 2, 16
NW = NC * NS
BE = 128
EPT = ((E // NW + BE - 1) // BE) * BE
E_PAD = EPT * NW
N_PAD = 10240
RZ = N_PAD // NS
RO = N // NS

_mesh = plsc.VectorSubcoreMesh(core_axis_name="c", subcore_axis_name="s")


@functools.lru_cache(maxsize=None)
def _make_deg():
  @functools.partial(
      pl.kernel,
      out_type=jax.ShapeDtypeStruct((NC, N, 16), jnp.float32),
      mesh=_mesh,
      scratch_types=[
          pltpu.VMEM((BE,), jnp.int32),
          pltpu.VMEM((BE, 16), jnp.float32),
          pltpu.VMEM_SHARED((N_PAD, 16), jnp.float32),
      ],
  )
  def deg_kernel(dst_hbm, zeros_hbm, ones_hbm, out_hbm, dst_i, ones_v, acc):
    cid = lax.axis_index("c")
    sid = lax.axis_index("s")
    t = cid * NS + sid
    pltpu.sync_copy(ones_hbm, ones_v)
    pltpu.sync_copy(zeros_hbm.at[pl.ds(sid * RZ, RZ)],
                    acc.at[pl.ds(sid * RZ, RZ)])
    plsc.subcore_barrier()

    def step(j, carry):
      off = pl.multiple_of(t * EPT + j * BE, BE)
      pltpu.sync_copy(dst_hbm.at[pl.ds(off, BE)], dst_i)
      pltpu.sync_copy(ones_v, acc.at[dst_i], add=True)
      return carry

    lax.fori_loop(0, EPT // BE, step, 0)
    plsc.subcore_barrier()
    pltpu.sync_copy(acc.at[pl.ds(sid * RO, RO)],
                    out_hbm.at[cid, pl.ds(sid * RO, RO)])

  return deg_kernel


@functools.lru_cache(maxsize=None)
def _make_agg():
  @functools.partial(
      pl.kernel,
      out_type=jax.ShapeDtypeStruct((NC, N, D), jnp.float32),
      mesh=_mesh,
      scratch_types=[
          pltpu.VMEM((BE,), jnp.int32),
          pltpu.VMEM((BE,), jnp.int32),
          pltpu.VMEM((BE, D), jnp.float32),
          pltpu.VMEM_SHARED((N_PAD, D), jnp.float32),
          pltpu.SemaphoreType.DMA,
      ],
  )
  def agg_kernel(src_hbm, dst_hbm, hs_hbm, zeros_hbm, out_hbm,
                 src_i, dst_i, rows_v, acc, sem):
    cid = lax.axis_index("c")
    sid = lax.axis_index("s")
    t = cid * NS + sid
    pltpu.sync_copy(zeros_hbm.at[pl.ds(sid * RZ, RZ)],
                    acc.at[pl.ds(sid * RZ, RZ)])
    plsc.subcore_barrier()

    def step(j, carry):
      off = pl.multiple_of(t * EPT + j * BE, BE)
      pltpu.sync_copy(src_hbm.at[pl.ds(off, BE)], src_i)
      pltpu.sync_copy(dst_hbm.at[pl.ds(off, BE)], dst_i)
      pltpu.async_copy(hs_hbm.at[src_i], rows_v, sem).wait()
      pltpu.sync_copy(rows_v, acc.at[dst_i], add=True)
      return carry

    lax.fori_loop(0, EPT // BE, step, 0)
    plsc.subcore_barrier()
    pltpu.sync_copy(acc.at[pl.ds(sid * RO, RO)],
                    out_hbm.at[cid, pl.ds(sid * RO, RO)])

  return agg_kernel


_R = 1000


def _ds_block(d0, d1):
  deg = d0[:, 0:1] + d1[:, 0:1] + 1.0
  return lax.rsqrt(deg)


def _stage1_body(x_ref, wp_ref, bp_ref, w1_ref, d0_ref, d1_ref, o_ref):
  ds = _ds_block(d0_ref[...], d1_ref[...])
  h0 = jnp.dot(x_ref[...], wp_ref[...],
               preferred_element_type=jnp.float32) + bp_ref[...]
  o_ref[...] = ds * jnp.dot(h0, w1_ref[...],
                            preferred_element_type=jnp.float32)


def _stage2_body(a0_ref, a1_ref, hs_ref, d0_ref, d1_ref, b1_ref, w2_ref,
                 o_ref):
  ds = _ds_block(d0_ref[...], d1_ref[...])
  pre = ds * (a0_ref[...] + a1_ref[...] + hs_ref[...]) + b1_ref[...]
  t = jnp.maximum(pre, 0.0)
  o_ref[...] = ds * jnp.dot(t, w2_ref[...],
                            preferred_element_type=jnp.float32)


def _stage3_body(a0_ref, a1_ref, hs_ref, d0_ref, d1_ref, b2_ref, o_ref):
  ds = _ds_block(d0_ref[...], d1_ref[...])
  o_ref[...] = ds * (a0_ref[...] + a1_ref[...] + hs_ref[...]) + b2_ref[...]


def _row_spec(w):
  return pl.BlockSpec((_R, w), lambda i: (i, 0))


def _full_spec(h, w):
  return pl.BlockSpec((h, w), lambda i: (0, 0))


@functools.lru_cache(maxsize=None)
def _make_stage1():
  return pl.pallas_call(
      _stage1_body,
      grid=(N // _R,),
      in_specs=[_row_spec(D), _full_spec(D, D), _full_spec(1, D),
                _full_spec(D, D), _row_spec(16), _row_spec(16)],
      out_specs=_row_spec(D),
      out_shape=jax.ShapeDtypeStruct((N, D), jnp.float32),
  )


@functools.lru_cache(maxsize=None)
def _make_stage2():
  return pl.pallas_call(
      _stage2_body,
      grid=(N // _R,),
      in_specs=[_row_spec(D), _row_spec(D), _row_spec(D),
                _row_spec(16), _row_spec(16), _full_spec(1, D),
                _full_spec(D, D)],
      out_specs=_row_spec(D),
      out_shape=jax.ShapeDtypeStruct((N, D), jnp.float32),
  )


@functools.lru_cache(maxsize=None)
def _make_stage3():
  return pl.pallas_call(
      _stage3_body,
      grid=(N // _R,),
      in_specs=[_row_spec(D), _row_spec(D), _row_spec(D),
                _row_spec(16), _row_spec(16), _full_spec(1, D)],
      out_specs=_row_spec(D),
      out_shape=jax.ShapeDtypeStruct((N, D), jnp.float32),
  )


def kernel(x, edge_index, W_pre, b_pre, W1, b1, W2, b2):
  ei = edge_index.astype(jnp.int32)
  pad = E_PAD - E
  src = jnp.concatenate([ei[0], jnp.zeros((pad,), jnp.int32)])
  dst = jnp.concatenate(
      [ei[1], N + (jnp.arange(pad, dtype=jnp.int32) % (N_PAD - N))])
  zeros16 = jnp.zeros((N_PAD, 16), jnp.float32)
  zerosD = jnp.zeros((N_PAD, D), jnp.float32)
  ones16 = jnp.ones((BE, 16), jnp.float32)

  degp = _make_deg()(dst, zeros16, ones16)
  d0, d1 = degp[0], degp[1]

  hs1 = _make_stage1()(x, W_pre, b_pre.reshape(1, D), W1, d0, d1)
  agg1 = _make_agg()(src, dst, hs1, zerosD)
  hs2 = _make_stage2()(agg1[0], agg1[1], hs1, d0, d1,
                       b1.reshape(1, D), W2)
  agg2 = _make_agg()(src, dst, hs2, zerosD)
  out = _make_stage3()(agg2[0], agg2[1], hs2, d0, d1, b2.reshape(1, D))
  return out


# speedup vs baseline: 10.0686x; 10.0686x over previous
import functools

import jax
import jax.numpy as jnp
from jax import lax
from jax.experimental import pallas as pl
from jax.experimental.pallas import tpu as pltpu
from jax.experimental.pallas import tpu_sc as plsc

N = 10000
E = 320000
D = 128
NC, NS = 2, 16
NW = NC * NS
BE = 128
EPT = ((E // NW + BE - 1) // BE) * BE
E_PAD = EPT * NW
N_PAD = 10240
RZ = N_PAD // NS
RO = N_PAD // NS

_mesh = plsc.VectorSubcoreMesh(core_axis_name="c", subcore_axis_name="s")


@functools.lru_cache(maxsize=None)
def _make_deg():
  @functools.partial(
      pl.kernel,
      out_type=jax.ShapeDtypeStruct((NC, N_PAD, D), jnp.float32),
      mesh=_mesh,
      scratch_types=[
          pltpu.VMEM((BE,), jnp.int32),
          pltpu.VMEM((BE, D), jnp.float32),
          pltpu.VMEM_SHARED((N_PAD, D), jnp.float32),
      ],
  )
  def deg_kernel(dst_hbm, zeros_hbm, ones_hbm, out_hbm, dst_i, ones_v, acc):
    cid = lax.axis_index("c")
    sid = lax.axis_index("s")
    t = cid * NS + sid
    pltpu.sync_copy(ones_hbm, ones_v)
    pltpu.sync_copy(zeros_hbm.at[pl.ds(sid * RZ, RZ)],
                    acc.at[pl.ds(sid * RZ, RZ)])
    plsc.subcore_barrier()

    def step(j, carry):
      off = pl.multiple_of(t * EPT + j * BE, BE)
      pltpu.sync_copy(dst_hbm.at[pl.ds(off, BE)], dst_i)
      pltpu.sync_copy(ones_v, acc.at[dst_i], add=True)
      return carry

    lax.fori_loop(0, EPT // BE, step, 0)
    plsc.subcore_barrier()
    pltpu.sync_copy(acc.at[pl.ds(sid * RO, RO)],
                    out_hbm.at[cid, pl.ds(sid * RO, RO)])

  return deg_kernel


@functools.lru_cache(maxsize=None)
def _make_agg():
  @functools.partial(
      pl.kernel,
      out_type=jax.ShapeDtypeStruct((NC, N_PAD, D), jnp.float32),
      mesh=_mesh,
      scratch_types=[
          pltpu.VMEM((BE,), jnp.int32),
          pltpu.VMEM((BE,), jnp.int32),
          pltpu.VMEM((BE, D), jnp.float32),
          pltpu.VMEM_SHARED((N_PAD, D), jnp.float32),
          pltpu.SemaphoreType.DMA,
      ],
  )
  def agg_kernel(src_hbm, dst_hbm, hs_hbm, zeros_hbm, out_hbm,
                 src_i, dst_i, rows_v, acc, sem):
    cid = lax.axis_index("c")
    sid = lax.axis_index("s")
    t = cid * NS + sid
    pltpu.sync_copy(zeros_hbm.at[pl.ds(sid * RZ, RZ)],
                    acc.at[pl.ds(sid * RZ, RZ)])
    plsc.subcore_barrier()

    def step(j, carry):
      off = pl.multiple_of(t * EPT + j * BE, BE)
      pltpu.sync_copy(src_hbm.at[pl.ds(off, BE)], src_i)
      pltpu.sync_copy(dst_hbm.at[pl.ds(off, BE)], dst_i)
      pltpu.async_copy(hs_hbm.at[src_i], rows_v, sem).wait()
      pltpu.sync_copy(rows_v, acc.at[dst_i], add=True)
      return carry

    lax.fori_loop(0, EPT // BE, step, 0)
    plsc.subcore_barrier()
    pltpu.sync_copy(acc.at[pl.ds(sid * RO, RO)],
                    out_hbm.at[cid, pl.ds(sid * RO, RO)])

  return agg_kernel


_R = 640


def _ds_block(d0, d1):
  deg = d0[:, 0:1] + d1[:, 0:1] + 1.0
  return lax.rsqrt(deg)


def _stage1_body(x_ref, wp_ref, bp_ref, w1_ref, d0_ref, d1_ref, o_ref):
  ds = _ds_block(d0_ref[...], d1_ref[...])
  h0 = jnp.dot(x_ref[...], wp_ref[...],
               preferred_element_type=jnp.float32) + bp_ref[...]
  o_ref[...] = ds * jnp.dot(h0, w1_ref[...],
                            preferred_element_type=jnp.float32)


def _stage2_body(a0_ref, a1_ref, hs_ref, d0_ref, d1_ref, b1_ref, w2_ref,
                 o_ref):
  ds = _ds_block(d0_ref[...], d1_ref[...])
  pre = ds * (a0_ref[...] + a1_ref[...] + hs_ref[...]) + b1_ref[...]
  t = jnp.maximum(pre, 0.0)
  o_ref[...] = ds * jnp.dot(t, w2_ref[...],
                            preferred_element_type=jnp.float32)


def _stage3_body(a0_ref, a1_ref, hs_ref, d0_ref, d1_ref, b2_ref, o_ref):
  ds = _ds_block(d0_ref[...], d1_ref[...])
  o_ref[...] = ds * (a0_ref[...] + a1_ref[...] + hs_ref[...]) + b2_ref[...]


def _row_spec(w):
  return pl.BlockSpec((_R, w), lambda i: (i, 0))


def _full_spec(h, w):
  return pl.BlockSpec((h, w), lambda i: (0, 0))


@functools.lru_cache(maxsize=None)
def _make_stage1():
  return pl.pallas_call(
      _stage1_body,
      grid=(N_PAD // _R,),
      in_specs=[_row_spec(D), _full_spec(D, D), _full_spec(1, D),
                _full_spec(D, D), _row_spec(D), _row_spec(D)],
      out_specs=_row_spec(D),
      out_shape=jax.ShapeDtypeStruct((N_PAD, D), jnp.float32),
  )


@functools.lru_cache(maxsize=None)
def _make_stage2():
  return pl.pallas_call(
      _stage2_body,
      grid=(N_PAD // _R,),
      in_specs=[_row_spec(D), _row_spec(D), _row_spec(D),
                _row_spec(D), _row_spec(D), _full_spec(1, D),
                _full_spec(D, D)],
      out_specs=_row_spec(D),
      out_shape=jax.ShapeDtypeStruct((N_PAD, D), jnp.float32),
  )


@functools.lru_cache(maxsize=None)
def _make_stage3():
  return pl.pallas_call(
      _stage3_body,
      grid=(N_PAD // _R,),
      in_specs=[_row_spec(D), _row_spec(D), _row_spec(D),
                _row_spec(D), _row_spec(D), _full_spec(1, D)],
      out_specs=_row_spec(D),
      out_shape=jax.ShapeDtypeStruct((N_PAD, D), jnp.float32),
  )


def kernel(x, edge_index, W_pre, b_pre, W1, b1, W2, b2):
  ei = edge_index.astype(jnp.int32)
  pad = E_PAD - E
  src = jnp.concatenate([ei[0], jnp.zeros((pad,), jnp.int32)])
  dst = jnp.concatenate(
      [ei[1], N + (jnp.arange(pad, dtype=jnp.int32) % (N_PAD - N))])
  zerosD = jnp.zeros((N_PAD, D), jnp.float32)
  onesD = jnp.ones((BE, D), jnp.float32)

  x_pad = jnp.concatenate([x, jnp.zeros((N_PAD - N, D), jnp.float32)])
  degp = _make_deg()(dst, zerosD, onesD)
  d0, d1 = degp[0], degp[1]

  hs1 = _make_stage1()(x_pad, W_pre, b_pre.reshape(1, D), W1, d0, d1)
  agg1 = _make_agg()(src, dst, hs1, zerosD)
  hs2 = _make_stage2()(agg1[0], agg1[1], hs1, d0, d1,
                       b1.reshape(1, D), W2)
  agg2 = _make_agg()(src, dst, hs2, zerosD)
  out = _make_stage3()(agg2[0], agg2[1], hs2, d0, d1, b2.reshape(1, D))
  return out[:N]
